# Initial kernel scaffold; baseline (speedup 1.0000x reference)
import jax
import jax.numpy as jnp
from jax.experimental import pallas as pl


def kernel(X, d, W, b):
    raise NotImplementedError("write your pallas kernel here")


# speedup vs baseline: 2.5574x; 2.5574x over previous
import functools

import jax
import jax.numpy as jnp
from jax import lax
from jax.experimental import pallas as pl
from jax.experimental.pallas import tpu as pltpu

E = 8
IN = 4096
HID = 4096
N = 8192
TM = 256
NP = N + E * TM
NT = NP // TM
TN = 1024
NN = HID // TN


def _mm_body(te_ref, x_ref, w_ref, b_ref, o_ref, wbf):
    m = pl.program_id(1)
    e = te_ref[m]
    prev = te_ref[jnp.maximum(m - 1, 0)]
    changed = jnp.logical_or(m == 0, e != prev)

    @pl.when(changed)
    def _():
        wbf[...] = w_ref[0].astype(jnp.bfloat16)

    acc = jnp.dot(x_ref[...], wbf[...], preferred_element_type=jnp.float32)
    o_ref[...] = acc + b_ref[0]


def _grouped_matmul(tile_expert, xs_bf, W, b):
    grid_spec = pltpu.PrefetchScalarGridSpec(
        num_scalar_prefetch=1,
        grid=(NN, NT),
        in_specs=[
            pl.BlockSpec((TM, IN), lambda n, m, te: (m, 0)),
            pl.BlockSpec((1, IN, TN), lambda n, m, te: (te[m], 0, n)),
            pl.BlockSpec((1, 1, TN), lambda n, m, te: (te[m], 0, n)),
        ],
        out_specs=pl.BlockSpec((TM, TN), lambda n, m, te: (m, n)),
        scratch_shapes=[pltpu.VMEM((IN, TN), jnp.bfloat16)],
    )
    return pl.pallas_call(
        _mm_body,
        grid_spec=grid_spec,
        out_shape=jax.ShapeDtypeStruct((NP, HID), jnp.float32),
        compiler_params=pltpu.CompilerParams(
            dimension_semantics=("arbitrary", "arbitrary"),
        ),
    )(tile_expert, xs_bf, W, b.reshape(E, 1, HID))


def kernel(X, d, W, b):
    counts = jnp.bincount(d, length=E)
    padded = (counts + TM - 1) & ~(TM - 1)
    ends = jnp.cumsum(padded)
    off = ends - padded
    starts = jnp.cumsum(counts) - counts
    perm = jnp.argsort(d, stable=True)
    es = d[perm]
    dst_sorted = off[es] + (jnp.arange(N, dtype=jnp.int32) - starts[es])
    dst = jnp.zeros((N,), jnp.int32).at[perm].set(dst_sorted.astype(jnp.int32))
    tile_expert = jnp.minimum(
        jnp.searchsorted(ends, jnp.arange(NT) * TM, side="right"), E - 1
    ).astype(jnp.int32)

    xb = X.astype(jnp.bfloat16)
    xs = jnp.zeros((NP, IN), jnp.bfloat16).at[dst].set(xb)

    ys = _grouped_matmul(tile_expert, xs, W, b)

    return ys[dst]
